# baseline (speedup 1.0000x reference)
import jax
import jax.numpy as jnp
from jax.experimental import pallas as pl
from jax.experimental.pallas import tpu as pltpu

_B, _N, _K = 32, 1024, 9
_NC = 40
_BIG = 3.0e38

_NET8 = [(0, 1), (2, 3), (4, 5), (6, 7),
         (0, 2), (1, 3), (4, 6), (5, 7),
         (1, 2), (5, 6), (0, 4), (3, 7),
         (1, 5), (2, 6),
         (1, 4), (3, 6),
         (2, 4), (3, 5),
         (3, 4)]


def _angle_gt(a, b):
    ax, ay, _ = a
    bx, by, _ = b
    ga = ay >= 0
    gb = by >= 0
    cross = ax * by - ay * bx
    same = jnp.logical_not(jnp.logical_xor(ga, gb))
    return (same & (cross < 0)) | (ga & jnp.logical_not(gb))


def _leaky(v):
    return jnp.maximum(v, jnp.float32(0.2) * v)


def _dot(a, b):
    return jax.lax.dot_general(a, b, (((1,), (0,)), ((), ())),
                               preferred_element_type=jnp.float32)


_BPP = 4


def _fwd(pts_ref, umbw_ref, umbb_ref,
         k0w_ref, k0b_ref, k1w_ref, k1b_ref, k2w_ref, k2b_ref,
         k3w_ref, k3b_ref, kow_ref, kob_ref,
         w1_ref, b1_ref, w2_ref, b2_ref, w3_ref, b3_ref,
         out_ref):
    for bi in range(_BPP):
        _one(bi, pts_ref, umbw_ref, umbb_ref,
             k0w_ref, k0b_ref, k1w_ref, k1b_ref, k2w_ref, k2b_ref,
             k3w_ref, k3b_ref, kow_ref, kob_ref,
             w1_ref, b1_ref, w2_ref, b2_ref, w3_ref, b3_ref, out_ref)


def _one(bi, pts_ref, umbw_ref, umbb_ref,
         k0w_ref, k0b_ref, k1w_ref, k1b_ref, k2w_ref, k2b_ref,
         k3w_ref, k3b_ref, kow_ref, kob_ref,
         w1_ref, b1_ref, w2_ref, b2_ref, w3_ref, b3_ref,
         out_ref):
    x = pts_ref[bi]
    sq = jnp.sum(x * x, axis=0, keepdims=True)
    hsq_col = (0.5 * sq).T
    g = jax.lax.dot_general(x, x, (((0,), (0,)), ((), ())),
                            preferred_element_type=jnp.float32)
    iota_m = jax.lax.broadcasted_iota(jnp.int32, (_N, 1), 0)
    iota_n = jax.lax.broadcasted_iota(jnp.int32, (1, _N), 1)

    imax = jnp.int32(0x7FFFFFFF)
    d = hsq_col - g
    s32 = jax.lax.bitcast_convert_type(d, jnp.int32)
    key = jnp.where(s32 < 0, jnp.int32(-2147483648) - s32, s32)
    key = (key & jnp.int32(-1024)) | iota_m

    key = jnp.where(iota_m == iota_n, imax, key)

    rels = []
    for _ in range(_K - 1):
        kmin = jnp.min(key, axis=0, keepdims=True)
        onehot = key == kmin
        sel = _dot(x, onehot.astype(jnp.float32))
        rels.append(sel - x)
        key = jnp.where(onehot, imax, key)

    items = [(r[0:1], r[1:2], r[2:3]) for r in rels]
    for i, j in _NET8:
        a, b = items[i], items[j]
        sw = _angle_gt(a, b)
        items[i] = tuple(jnp.where(sw, bb, aa) for aa, bb in zip(a, b))
        items[j] = tuple(jnp.where(sw, aa, bb) for aa, bb in zip(a, b))

    umbw = umbw_ref[...]
    umbb = umbb_ref[...]
    normal = None
    for t in range(8):
        ax, ay, az = items[t]
        bx, by, bz = items[(t + 1) % 8]
        nx = ay * bz - az * by
        ny = az * bx - ax * bz
        nz = ax * by - ay * bx
        nn = jnp.sqrt(nx * nx + ny * ny + nz * nz) + jnp.float32(1e-8)
        nx, ny, nz = nx / nn, ny / nn, nz / nn
        flip = nz < 0
        nx = jnp.where(flip, -nx, nx)
        ny = jnp.where(flip, -ny, ny)
        nz = jnp.where(flip, -nz, nz)
        third = jnp.float32(1.0 / 3.0)
        cx = (ax + bx) * third
        cy = (ay + by) * third
        cz = (az + bz) * third
        pos = nx * cx + ny * cy + nz * cz
        feat = jnp.concatenate([cx, cy, cz, nx, ny, nz, pos], axis=0)
        h = jnp.maximum(_dot(umbw, feat) + umbb, 0.0)
        normal = h if normal is None else normal + h

    h = jnp.concatenate([x, normal], axis=0)
    h = _leaky(_dot(k0w_ref[...], h) + k0b_ref[...])
    h = _leaky(_dot(k1w_ref[...], h) + k1b_ref[...])
    h = _leaky(_dot(k2w_ref[...], h) + k2b_ref[...])
    h = _leaky(_dot(k3w_ref[...], h) + k3b_ref[...])
    z = _dot(kow_ref[...], h) + kob_ref[...]
    fp = jnp.max(z, axis=1, keepdims=True).T

    y = _leaky(_dot(fp, w1_ref[...]) + b1_ref[...])
    y = _leaky(_dot(y, w2_ref[...]) + b2_ref[...])
    lg = _dot(y, w3_ref[...]) + b3_ref[...]
    m = jnp.max(lg, axis=1, keepdims=True)
    e = jnp.exp(lg - m)
    s = jnp.sum(e, axis=1, keepdims=True)
    out_ref[bi] = lg - m - jnp.log(s)


def kernel(points, params):
    f32 = jnp.float32

    def col(v):
        return v.reshape(-1, 1).astype(f32)

    def row(v):
        return v.reshape(1, -1).astype(f32)

    umbw = params["umb_mlp"]["w"].T
    umbb = col(params["umb_mlp"]["b"])

    khw, khb = [], []
    for lin, bn in zip(params["kh"], params["kh_bn"]):
        khw.append((lin["w"] * bn["g"][None, :]).T)
        khb.append(col(lin["b"] * bn["g"] + bn["b"]))
    kow = params["kh_out"]["w"].T
    kob = col(params["kh_out"]["b"])

    w1 = params["fc1"]["w"] * params["bn1"]["g"][None, :]
    b1 = row(params["fc1"]["b"] * params["bn1"]["g"] + params["bn1"]["b"])
    w2 = params["fc2"]["w"] * params["bn2"]["g"][None, :]
    b2 = row(params["fc2"]["b"] * params["bn2"]["g"] + params["bn2"]["b"])
    w3 = params["fc3"]["w"]
    b3 = row(params["fc3"]["b"])

    def fixed(a):
        return pl.BlockSpec(a.shape, lambda b: (0,) * a.ndim)

    args = [umbw, umbb, khw[0], khb[0], khw[1], khb[1], khw[2], khb[2],
            khw[3], khb[3], kow, kob, w1, b1, w2, b2, w3, b3]

    out = pl.pallas_call(
        _fwd,
        grid=(_B // _BPP,),
        in_specs=[pl.BlockSpec((_BPP, 3, _N), lambda b: (b, 0, 0))]
                 + [fixed(a) for a in args],
        out_specs=pl.BlockSpec((_BPP, 1, _NC), lambda b: (b, 0, 0)),
        out_shape=jax.ShapeDtypeStruct((_B, 1, _NC), f32),
        compiler_params=pltpu.CompilerParams(
            dimension_semantics=("arbitrary",)),
    )(points, *args)
    return out.reshape(_B, _NC)


# speedup vs baseline: 1.3547x; 1.3547x over previous
import jax
import jax.numpy as jnp
from jax.experimental import pallas as pl
from jax.experimental.pallas import tpu as pltpu

_B, _N, _K = 32, 1024, 9
_NC = 40
_BPP = 2

_NET8 = [(0, 1), (2, 3), (4, 5), (6, 7),
         (0, 2), (1, 3), (4, 6), (5, 7),
         (1, 2), (5, 6), (0, 4), (3, 7),
         (1, 5), (2, 6),
         (1, 4), (3, 6),
         (2, 4), (3, 5),
         (3, 4)]


def _angle_gt(a, b):
    ax, ay, _ = a
    bx, by, _ = b
    ga = ay >= 0
    gb = by >= 0
    cross = ax * by - ay * bx
    same = jnp.logical_not(jnp.logical_xor(ga, gb))
    return (same & (cross < 0)) | (ga & jnp.logical_not(gb))


def _leaky(v):
    return jnp.maximum(v, jnp.float32(0.2) * v)


def _dot(a, b):
    return jax.lax.dot_general(a, b, (((1,), (0,)), ((), ())),
                               preferred_element_type=jnp.float32)


def _topk_rels(x):
    sq = jnp.sum(x * x, axis=0, keepdims=True)
    hsq_col = (0.5 * sq).T
    g = jax.lax.dot_general(x, x, (((0,), (0,)), ((), ())),
                            preferred_element_type=jnp.float32)
    iota_m = jax.lax.broadcasted_iota(jnp.int32, (_N, 1), 0)
    iota_n = jax.lax.broadcasted_iota(jnp.int32, (1, _N), 1)

    imax = jnp.int32(0x7FFFFFFF)
    d = hsq_col - g
    s32 = jax.lax.bitcast_convert_type(d, jnp.int32)
    key = jnp.where(s32 < 0, jnp.int32(-2147483648) - s32, s32)
    key = (key & jnp.int32(-1024)) | iota_m

    key = jnp.where(iota_m == iota_n, imax, key)

    rels = []
    for _ in range(_K - 1):
        kmin = jnp.min(key, axis=0, keepdims=True)
        onehot = key == kmin
        sel = _dot(x, onehot.astype(jnp.float32))
        rels.append(sel - x)
        key = jnp.where(onehot, imax, key)
    return rels


def _fwd(pts_ref, umbw_ref, umbb_ref,
         k0w_ref, k0b_ref, k1w_ref, k1b_ref, k2w_ref, k2b_ref,
         k3w_ref, k3b_ref, kow_ref, kob_ref,
         w1_ref, b1_ref, w2_ref, b2_ref, w3_ref, b3_ref,
         out_ref):
    xs = []
    rels_per_b = []
    for bi in range(_BPP):
        x = pts_ref[bi]
        xs.append(x)
        rels_per_b.append(_topk_rels(x))

    xw = jnp.concatenate(xs, axis=1)
    rels = [jnp.concatenate([rb[k] for rb in rels_per_b], axis=1)
            for k in range(_K - 1)]

    items = [(r[0:1], r[1:2], r[2:3]) for r in rels]
    for i, j in _NET8:
        a, b = items[i], items[j]
        sw = _angle_gt(a, b)
        items[i] = tuple(jnp.where(sw, bb, aa) for aa, bb in zip(a, b))
        items[j] = tuple(jnp.where(sw, aa, bb) for aa, bb in zip(a, b))

    umbw = umbw_ref[...]
    umbb = umbb_ref[...]
    normal = None
    for t in range(8):
        ax, ay, az = items[t]
        bx, by, bz = items[(t + 1) % 8]
        nx = ay * bz - az * by
        ny = az * bx - ax * bz
        nz = ax * by - ay * bx
        nn = jnp.sqrt(nx * nx + ny * ny + nz * nz) + jnp.float32(1e-8)
        nx, ny, nz = nx / nn, ny / nn, nz / nn
        flip = nz < 0
        nx = jnp.where(flip, -nx, nx)
        ny = jnp.where(flip, -ny, ny)
        nz = jnp.where(flip, -nz, nz)
        third = jnp.float32(1.0 / 3.0)
        cx = (ax + bx) * third
        cy = (ay + by) * third
        cz = (az + bz) * third
        pos = nx * cx + ny * cy + nz * cz
        feat = jnp.concatenate([cx, cy, cz, nx, ny, nz, pos], axis=0)
        h = jnp.maximum(_dot(umbw, feat) + umbb, 0.0)
        normal = h if normal is None else normal + h

    h = jnp.concatenate([xw, normal], axis=0)
    h = _leaky(_dot(k0w_ref[...], h) + k0b_ref[...])
    h = _leaky(_dot(k1w_ref[...], h) + k1b_ref[...])
    h = _leaky(_dot(k2w_ref[...], h) + k2b_ref[...])
    h = _leaky(_dot(k3w_ref[...], h) + k3b_ref[...])
    z = _dot(kow_ref[...], h) + kob_ref[...]
    fp = jnp.concatenate(
        [jnp.max(z[:, bi * _N:(bi + 1) * _N], axis=1, keepdims=True).T
         for bi in range(_BPP)], axis=0)

    y = _leaky(_dot(fp, w1_ref[...]) + b1_ref[...])
    y = _leaky(_dot(y, w2_ref[...]) + b2_ref[...])
    lg = _dot(y, w3_ref[...]) + b3_ref[...]
    m = jnp.max(lg, axis=1, keepdims=True)
    e = jnp.exp(lg - m)
    s = jnp.sum(e, axis=1, keepdims=True)
    out = lg - m - jnp.log(s)
    out_ref[...] = out.reshape(_BPP, 1, _NC)


def kernel(points, params):
    f32 = jnp.float32

    def col(v):
        return v.reshape(-1, 1).astype(f32)

    def row(v):
        return v.reshape(1, -1).astype(f32)

    umbw = params["umb_mlp"]["w"].T
    umbb = col(params["umb_mlp"]["b"])

    khw, khb = [], []
    for lin, bn in zip(params["kh"], params["kh_bn"]):
        khw.append((lin["w"] * bn["g"][None, :]).T)
        khb.append(col(lin["b"] * bn["g"] + bn["b"]))
    kow = params["kh_out"]["w"].T
    kob = col(params["kh_out"]["b"])

    w1 = params["fc1"]["w"] * params["bn1"]["g"][None, :]
    b1 = row(params["fc1"]["b"] * params["bn1"]["g"] + params["bn1"]["b"])
    w2 = params["fc2"]["w"] * params["bn2"]["g"][None, :]
    b2 = row(params["fc2"]["b"] * params["bn2"]["g"] + params["bn2"]["b"])
    w3 = params["fc3"]["w"]
    b3 = row(params["fc3"]["b"])

    def fixed(a):
        return pl.BlockSpec(a.shape, lambda b: (0,) * a.ndim)

    args = [umbw, umbb, khw[0], khb[0], khw[1], khb[1], khw[2], khb[2],
            khw[3], khb[3], kow, kob, w1, b1, w2, b2, w3, b3]

    out = pl.pallas_call(
        _fwd,
        grid=(_B // _BPP,),
        in_specs=[pl.BlockSpec((_BPP, 3, _N), lambda b: (b, 0, 0))]
                 + [fixed(a) for a in args],
        out_specs=pl.BlockSpec((_BPP, 1, _NC), lambda b: (b, 0, 0)),
        out_shape=jax.ShapeDtypeStruct((_B, 1, _NC), f32),
        compiler_params=pltpu.CompilerParams(
            dimension_semantics=("arbitrary",)),
    )(points, *args)
    return out.reshape(_B, _NC)


# speedup vs baseline: 1.4197x; 1.0480x over previous
import jax
import jax.numpy as jnp
from jax.experimental import pallas as pl
from jax.experimental.pallas import tpu as pltpu

_B, _N, _K = 32, 1024, 9
_NC = 40
_BPP = 4

_NET8 = [(0, 1), (2, 3), (4, 5), (6, 7),
         (0, 2), (1, 3), (4, 6), (5, 7),
         (1, 2), (5, 6), (0, 4), (3, 7),
         (1, 5), (2, 6),
         (1, 4), (3, 6),
         (2, 4), (3, 5),
         (3, 4)]


def _angle_gt(a, b):
    ax, ay, _ = a
    bx, by, _ = b
    ga = ay >= 0
    gb = by >= 0
    cross = ax * by - ay * bx
    same = jnp.logical_not(jnp.logical_xor(ga, gb))
    return (same & (cross < 0)) | (ga & jnp.logical_not(gb))


def _leaky(v):
    return jnp.maximum(v, jnp.float32(0.2) * v)


def _dot(a, b):
    return jax.lax.dot_general(a, b, (((1,), (0,)), ((), ())),
                               preferred_element_type=jnp.float32)


def _topk_rels(x):
    sq = jnp.sum(x * x, axis=0, keepdims=True)
    hsq_col = (0.5 * sq).T
    g = jax.lax.dot_general(x, x, (((0,), (0,)), ((), ())),
                            preferred_element_type=jnp.float32)
    iota_m = jax.lax.broadcasted_iota(jnp.int32, (_N, 1), 0)
    iota_n = jax.lax.broadcasted_iota(jnp.int32, (1, _N), 1)

    imax = jnp.int32(0x7FFFFFFF)
    d = hsq_col - g
    s32 = jax.lax.bitcast_convert_type(d, jnp.int32)
    key = jnp.where(s32 < 0, jnp.int32(-2147483648) - s32, s32)
    key = (key & jnp.int32(-1024)) | iota_m

    key = jnp.where(iota_m == iota_n, imax, key)

    rels = []
    for _ in range(_K - 1):
        kmin = jnp.min(key, axis=0, keepdims=True)
        onehot = key == kmin
        sel = _dot(x, onehot.astype(jnp.float32))
        rels.append(sel - x)
        key = jnp.where(onehot, imax, key)
    return rels


def _fwd(pts_ref, umbw_ref, umbb_ref,
         k0w_ref, k0b_ref, k1w_ref, k1b_ref, k2w_ref, k2b_ref,
         k3w_ref, k3b_ref, kow_ref, kob_ref,
         w1_ref, b1_ref, w2_ref, b2_ref, w3_ref, b3_ref,
         out_ref):
    xs = []
    rels_per_b = []
    for bi in range(_BPP):
        x = pts_ref[bi]
        xs.append(x)
        rels_per_b.append(_topk_rels(x))

    xw = jnp.concatenate(xs, axis=1)
    rels = [jnp.concatenate([rb[k] for rb in rels_per_b], axis=1)
            for k in range(_K - 1)]

    items = [(r[0:1], r[1:2], r[2:3]) for r in rels]
    for i, j in _NET8:
        a, b = items[i], items[j]
        sw = _angle_gt(a, b)
        items[i] = tuple(jnp.where(sw, bb, aa) for aa, bb in zip(a, b))
        items[j] = tuple(jnp.where(sw, aa, bb) for aa, bb in zip(a, b))

    umbw = umbw_ref[...]
    umbb = umbb_ref[...]
    normal = None
    for t in range(8):
        ax, ay, az = items[t]
        bx, by, bz = items[(t + 1) % 8]
        nx = ay * bz - az * by
        ny = az * bx - ax * bz
        nz = ax * by - ay * bx
        nn = jnp.sqrt(nx * nx + ny * ny + nz * nz) + jnp.float32(1e-8)
        nx, ny, nz = nx / nn, ny / nn, nz / nn
        flip = nz < 0
        nx = jnp.where(flip, -nx, nx)
        ny = jnp.where(flip, -ny, ny)
        nz = jnp.where(flip, -nz, nz)
        third = jnp.float32(1.0 / 3.0)
        cx = (ax + bx) * third
        cy = (ay + by) * third
        cz = (az + bz) * third
        pos = nx * cx + ny * cy + nz * cz
        feat = jnp.concatenate([cx, cy, cz, nx, ny, nz, pos], axis=0)
        h = jnp.maximum(_dot(umbw, feat) + umbb, 0.0)
        normal = h if normal is None else normal + h

    h = jnp.concatenate([xw, normal], axis=0)
    h = _leaky(_dot(k0w_ref[...], h) + k0b_ref[...])
    h = _leaky(_dot(k1w_ref[...], h) + k1b_ref[...])
    h = _leaky(_dot(k2w_ref[...], h) + k2b_ref[...])
    h = _leaky(_dot(k3w_ref[...], h) + k3b_ref[...])
    z = _dot(kow_ref[...], h) + kob_ref[...]
    fp = jnp.concatenate(
        [jnp.max(z[:, bi * _N:(bi + 1) * _N], axis=1, keepdims=True).T
         for bi in range(_BPP)], axis=0)

    y = _leaky(_dot(fp, w1_ref[...]) + b1_ref[...])
    y = _leaky(_dot(y, w2_ref[...]) + b2_ref[...])
    lg = _dot(y, w3_ref[...]) + b3_ref[...]
    m = jnp.max(lg, axis=1, keepdims=True)
    e = jnp.exp(lg - m)
    s = jnp.sum(e, axis=1, keepdims=True)
    out = lg - m - jnp.log(s)
    out_ref[...] = out.reshape(_BPP, 1, _NC)


def kernel(points, params):
    f32 = jnp.float32

    def col(v):
        return v.reshape(-1, 1).astype(f32)

    def row(v):
        return v.reshape(1, -1).astype(f32)

    umbw = params["umb_mlp"]["w"].T
    umbb = col(params["umb_mlp"]["b"])

    khw, khb = [], []
    for lin, bn in zip(params["kh"], params["kh_bn"]):
        khw.append((lin["w"] * bn["g"][None, :]).T)
        khb.append(col(lin["b"] * bn["g"] + bn["b"]))
    kow = params["kh_out"]["w"].T
    kob = col(params["kh_out"]["b"])

    w1 = params["fc1"]["w"] * params["bn1"]["g"][None, :]
    b1 = row(params["fc1"]["b"] * params["bn1"]["g"] + params["bn1"]["b"])
    w2 = params["fc2"]["w"] * params["bn2"]["g"][None, :]
    b2 = row(params["fc2"]["b"] * params["bn2"]["g"] + params["bn2"]["b"])
    w3 = params["fc3"]["w"]
    b3 = row(params["fc3"]["b"])

    def fixed(a):
        return pl.BlockSpec(a.shape, lambda b: (0,) * a.ndim)

    args = [umbw, umbb, khw[0], khb[0], khw[1], khb[1], khw[2], khb[2],
            khw[3], khb[3], kow, kob, w1, b1, w2, b2, w3, b3]

    out = pl.pallas_call(
        _fwd,
        grid=(_B // _BPP,),
        in_specs=[pl.BlockSpec((_BPP, 3, _N), lambda b: (b, 0, 0))]
                 + [fixed(a) for a in args],
        out_specs=pl.BlockSpec((_BPP, 1, _NC), lambda b: (b, 0, 0)),
        out_shape=jax.ShapeDtypeStruct((_B, 1, _NC), f32),
        compiler_params=pltpu.CompilerParams(
            dimension_semantics=("arbitrary",)),
    )(points, *args)
    return out.reshape(_B, _NC)
